# Initial kernel scaffold; baseline (speedup 1.0000x reference)
import jax
import jax.numpy as jnp
from jax.experimental import pallas as pl


def kernel(dense_batch, embeddings, primitives_raw, identity):
    raise NotImplementedError("write your pallas kernel here")


# speedup vs baseline: 2.6222x; 2.6222x over previous
import functools

import jax
import jax.numpy as jnp
from jax import lax
from jax.experimental import pallas as pl
from jax.experimental.pallas import tpu as pltpu
from jax.experimental.pallas import tpu_sc as plsc

DIM = 1024
D2 = DIM // 2
NTOK = 4 * 8192


_IDX_R, _IDX_C = 256, 128


def _table_index_body(tt_ref, tv_ref, np_ref, emb_ref, primT_ref, id_ref,
                      bigT_ref, g_ref):
    id8 = jnp.broadcast_to(id_ref[...], (8, D2))
    p0t = primT_ref[0]
    p1t = primT_ref[1]
    x1 = jnp.dot(id8, p0t, preferred_element_type=jnp.float32)
    y1 = jnp.dot(id8, p1t, preferred_element_type=jnp.float32)
    x2 = jnp.dot(x1, p0t, preferred_element_type=jnp.float32)
    y2 = jnp.dot(y1, p0t, preferred_element_type=jnp.float32)
    rid = lax.broadcasted_iota(jnp.int32, (8, D2), 0)
    p8 = jnp.where(rid < 2, id8,
         jnp.where(rid == 2, x1,
         jnp.where(rid == 3, y1,
         jnp.where(rid == 4, x2,
         jnp.where(rid == 5, y2, jnp.zeros_like(id8))))))

    bigT_ref[:, :, D2:] = jnp.broadcast_to(p8[None, :, :], (24, 8, D2))
    bigT_ref[0:8, :, 0:D2] = jnp.broadcast_to(p8[:, None, :], (8, 8, D2))
    bigT_ref[8:24, :, 0:D2] = jnp.broadcast_to(emb_ref[...][:, None, :],
                                               (16, 8, D2))

    tt = tt_ref[...]
    tv = tv_ref[...]
    npos = np_ref[...]
    present = [jnp.sum(jnp.where(npos == v, 1, 0)) > 0 for v in range(6)]
    db = jnp.full((_IDX_R, _IDX_C), -1, jnp.int32)
    for v in range(5, -1, -1):
        db = jnp.where(jnp.logical_and(present[v], tv <= v), v, db)
    mp = jnp.int32(-1)
    for v in range(6):
        mp = jnp.where(present[v], jnp.int32(v), mp)
    db = jnp.where(db >= 0, db, mp)

    c = jnp.full((_IDX_R, _IDX_C), 19, jnp.int32)
    c = jnp.where(tt == 0, 8, c)
    c = jnp.where(tt == 1, 9 + tv, c)
    c = jnp.where(tt == 2, 13 + tv, c)
    c = jnp.where(tt == 4, db, c)
    g_ref[...] = c * 8 + npos


def _build_table_and_index(tt, tv, npos, emb16, primT, identity):
    return pl.pallas_call(
        _table_index_body,
        out_shape=[
            jax.ShapeDtypeStruct((24, 8, DIM), jnp.float32),
            jax.ShapeDtypeStruct((_IDX_R, _IDX_C), jnp.int32),
        ],
    )(tt, tv, npos, emb16, primT, identity)


_NC = 2
_NS = 16
_NW = _NC * _NS
_BPW = NTOK // _NW
_K = 64
_NCHUNK = _BPW // _K


def _gather_body(table_hbm, idx_hbm, out_hbm, idx_v, rows_v, sem):
    wid = lax.axis_index("s") * _NC + lax.axis_index("c")
    base = wid * _BPW
    pltpu.sync_copy(idx_hbm.at[wid], idx_v)
    for ck in range(_NCHUNK):
        pltpu.async_copy(table_hbm.at[idx_v.at[ck]], rows_v, sem).wait()
        pltpu.sync_copy(rows_v, out_hbm.at[pl.ds(base + ck * _K, _K)])


@functools.cache
def _gather_rows_kernel():
    return functools.partial(
        pl.kernel,
        mesh=plsc.VectorSubcoreMesh(core_axis_name="c", subcore_axis_name="s"),
        out_type=jax.ShapeDtypeStruct((NTOK, DIM), jnp.float32),
        scratch_types=[
            pltpu.VMEM((_NCHUNK, _K), jnp.int32),
            pltpu.VMEM((_K, DIM), jnp.float32),
            pltpu.SemaphoreType.DMA,
        ],
    )(_gather_body)


def kernel(dense_batch, embeddings, primitives_raw, identity):
    f32 = jnp.float32
    X = jnp.tril(primitives_raw.astype(f32))
    A = X - jnp.swapaxes(X, -1, -2)
    I = jnp.eye(D2, dtype=f32)
    P = jnp.linalg.solve(I - 0.5 * A, I + 0.5 * A)
    primT = jnp.swapaxes(P, -1, -2)

    emb16 = jnp.pad(embeddings.astype(f32), ((0, 5), (0, 0)))
    tt = dense_batch[0].reshape(_IDX_R, _IDX_C)
    tv = dense_batch[1].reshape(_IDX_R, _IDX_C)
    npos = dense_batch[2].reshape(_IDX_R, _IDX_C)

    bigT3, g = _build_table_and_index(tt, tv, npos, emb16, primT,
                                      identity.astype(f32))
    bigT = bigT3.reshape(24 * 8, DIM)
    gidx = g.reshape(_NW, _NCHUNK, _K)

    out = _gather_rows_kernel()(bigT, gidx)
    return out.reshape(4, 8192, DIM)
